# Initial kernel scaffold; baseline (speedup 1.0000x reference)
import jax
import jax.numpy as jnp
from jax.experimental import pallas as pl


def kernel(predictions, rule_weights, A_in_0, A_out_0, A_in_1, A_out_1, A_in_2, A_out_2):
    raise NotImplementedError("write your pallas kernel here")


# speedup vs baseline: 22.0777x; 22.0777x over previous
import functools

import jax
import jax.numpy as jnp
from jax import lax
from jax.experimental import pallas as pl
from jax.experimental.pallas import tpu as pltpu, tpu_sc as plsc

_N_ATOMS = 1000000
_N_GROUND = 500000
_BODY_LEN = 4
_HEAD_LEN = 2
_LANES = 16

_TOTAL_G = _N_GROUND // _LANES
_NW = 32
_GPW = 980
_CG = 140
_NCH = _GPW // _CG
_BC = _CG * _LANES * _BODY_LEN
_HC = _CG * _LANES * _HEAD_LEN


def _make_sc_kernel():
    mesh = plsc.VectorSubcoreMesh(core_axis_name="c", subcore_axis_name="s")

    @functools.partial(
        pl.kernel,
        mesh=mesh,
        out_type=jax.ShapeDtypeStruct((3, _NW, _LANES), jnp.float32),
        compiler_params=pltpu.CompilerParams(needs_layout_passes=False),
        scratch_types=[
            pltpu.VMEM_SHARED((_N_ATOMS,), jnp.float32),
            pltpu.VMEM((_BC,), jnp.int32),
            pltpu.VMEM((_BC,), jnp.float32),
            pltpu.VMEM((_HC,), jnp.int32),
            pltpu.VMEM((_HC,), jnp.float32),
            pltpu.VMEM((_LANES,), jnp.float32),
            pltpu.SemaphoreType.DMA,
            pltpu.SemaphoreType.DMA,
        ],
    )
    def sc_kernel(pred_hbm, ain0, aout0, ain1, aout1, ain2, aout2, out_hbm,
                  spmem, bidx, bval, hidx, hval, stage, sem_b, sem_h):
        cid = lax.axis_index("c")
        sid = lax.axis_index("s")
        wid = sid * 2 + cid

        @pl.when(sid == 0)
        def _stage_table():
            pltpu.sync_copy(pred_hbm, spmem)

        plsc.subcore_barrier()

        base_g = jnp.minimum(wid * _GPW, _TOTAL_G - _GPW)
        skip = wid * _GPW - base_g

        for r, (ain, aout) in enumerate(
                ((ain0, aout0), (ain1, aout1), (ain2, aout2))):
            def chunk_body(c, acc, ain=ain, aout=aout):
                g0 = base_g + c * _CG
                pltpu.sync_copy(
                    ain.at[pl.ds(g0 * (_LANES * _BODY_LEN), _BC)], bidx)
                pltpu.sync_copy(
                    aout.at[pl.ds(g0 * (_LANES * _HEAD_LEN), _HC)], hidx)
                cp_b = pltpu.async_copy(spmem.at[bidx], bval, sem_b)
                cp_h = pltpu.async_copy(spmem.at[hidx], hval, sem_h)
                cp_b.wait()
                cp_h.wait()

                def group_body(g, a):
                    i16 = lax.iota(jnp.int32, _LANES)
                    b0 = g * (_LANES * _BODY_LEN)
                    h0 = g * (_LANES * _HEAD_LEN)
                    t = plsc.load_gather(bval, [i16 * _BODY_LEN + b0])
                    for j in range(1, _BODY_LEN):
                        t = t * plsc.load_gather(
                            bval, [i16 * _BODY_LEN + (b0 + j)])
                    for j in range(_HEAD_LEN):
                        t = t * (jnp.float32(1.0) - plsc.load_gather(
                            hval, [i16 * _HEAD_LEN + (h0 + j)]))
                    n = c * _CG + g
                    f = jnp.where(n >= skip, jnp.float32(1.0),
                                  jnp.float32(0.0))
                    return a + t * f

                return lax.fori_loop(0, _CG, group_body, acc)

            acc = lax.fori_loop(0, _NCH, chunk_body,
                                jnp.zeros((_LANES,), jnp.float32))
            stage[...] = acc
            pltpu.sync_copy(stage, out_hbm.at[r, wid])

    return sc_kernel


_SC_KERNEL = _make_sc_kernel()


def kernel(predictions, rule_weights, A_in_0, A_out_0, A_in_1, A_out_1,
           A_in_2, A_out_2):
    ains = [a.reshape(-1).astype(jnp.int32)
            for a in (A_in_0, A_in_1, A_in_2)]
    aouts = [a.reshape(-1).astype(jnp.int32)
             for a in (A_out_0, A_out_1, A_out_2)]
    partials = _SC_KERNEL(predictions, ains[0], aouts[0], ains[1], aouts[1],
                          ains[2], aouts[2])
    s = partials.sum(axis=(1, 2))
    return jnp.sum(rule_weights * s) / jnp.float32(_N_GROUND)
